# baseline (speedup 1.0000x reference)
import functools

import jax
import jax.numpy as jnp
from jax import lax
from jax.experimental import pallas as pl
from jax.experimental.pallas import tpu as pltpu
from jax.experimental.pallas import tpu_sc as plsc

D_MODEL = 64
SCALE = 8.0

_INFO = plsc.get_sparse_core_info()
NC = _INFO.num_cores
NS = _INFO.num_subcores
NW = NC * NS
CHUNK = 128
NBUF = 2
KCOL = 12288


@functools.lru_cache(maxsize=None)
def _build_dup(vocab):
    ghalf = (vocab + 2 * KCOL - 1) // (2 * KCOL)
    v2 = ghalf * KCOL
    nblk_in = (vocab + KCOL - 1) // KCOL

    def dup(lo_ref, hi_ref, t4_ref):
        t4_ref[...] = (
            jnp.concatenate([lo_ref[...].T, hi_ref[...].T], axis=1) * SCALE
        )

    return pl.pallas_call(
        dup,
        grid=(ghalf,),
        in_specs=[
            pl.BlockSpec((D_MODEL, KCOL), lambda g: (0, g)),
            pl.BlockSpec(
                (D_MODEL, KCOL),
                lambda g: (0, jnp.minimum(g + ghalf, nblk_in - 1)),
            ),
        ],
        out_specs=pl.BlockSpec((KCOL, 2 * D_MODEL), lambda g: (g, 0)),
        out_shape=jax.ShapeDtypeStruct((v2, 2 * D_MODEL), jnp.float32),
    ), v2


@functools.lru_cache(maxsize=None)
def _build_gather(n_blocks, vpad):
    mesh = plsc.VectorSubcoreMesh(core_axis_name="c", subcore_axis_name="s")

    @functools.partial(
        pl.kernel,
        mesh=mesh,
        out_type=jax.ShapeDtypeStruct(
            (NW, n_blocks, CHUNK, D_MODEL), jnp.float32
        ),
        scratch_types=[
            pltpu.VMEM((n_blocks, CHUNK), jnp.int32),
            pltpu.VMEM((n_blocks, CHUNK), jnp.int32),
            [pltpu.VMEM((CHUNK, 2 * D_MODEL), jnp.float32) for _ in range(NBUF)],
            [pltpu.VMEM((CHUNK, D_MODEL), jnp.float32) for _ in range(NBUF)],
            [pltpu.SemaphoreType.DMA for _ in range(NBUF)],
            [pltpu.SemaphoreType.DMA for _ in range(NBUF)],
        ],
    )
    def gather(x_hbm, par_hbm, t4_hbm, out_hbm, idx_v, par_v, ins, outs, gsems, ssems):
        wid = lax.axis_index("s") * NC + lax.axis_index("c")
        pltpu.sync_copy(x_hbm.at[wid], idx_v)
        pltpu.sync_copy(par_hbm.at[wid], par_v)

        for b in range(NBUF):
            pltpu.async_copy(t4_hbm.at[idx_v.at[b]], ins[b], gsems[b])

        def step(j, _):
            for b in range(NBUF):
                blk = j + b
                pltpu.make_async_copy(
                    t4_hbm.at[idx_v.at[blk]], ins[b], gsems[b]
                ).wait()

                @pl.when(j > 0)
                def _():
                    pltpu.make_async_copy(
                        outs[b], out_hbm.at[wid, blk], ssems[b]
                    ).wait()

                def crow(r16, _):
                    r = r16 * 16
                    pv = par_v[blk, pl.ds(r, 16)]
                    for rr in range(16):
                        base = pv[rr] * D_MODEL
                        for q in range(D_MODEL // 16):
                            seg = pl.ds(q * 16, 16)
                            outs[b][r + rr, seg] = ins[b][
                                r + rr, pl.ds(base + q * 16, 16)
                            ]
                    return ()

                lax.fori_loop(0, CHUNK // 16, crow, ())

                pltpu.async_copy(outs[b], out_hbm.at[wid, blk], ssems[b])

                @pl.when(j + NBUF < n_blocks)
                def _():
                    pltpu.async_copy(
                        t4_hbm.at[idx_v.at[blk + NBUF]], ins[b], gsems[b]
                    )
            return ()

        lax.fori_loop(0, n_blocks // NBUF, lambda i, c: step(i * NBUF, c), ())

        for b in range(NBUF):
            pltpu.make_async_copy(
                outs[b], out_hbm.at[wid, n_blocks - NBUF + b], ssems[b]
            ).wait()

    return gather


def kernel(x, table):
    n_batch, seq_len = x.shape
    vocab, d = table.shape
    total = n_batch * seq_len
    per_w = total // NW
    n_blocks = per_w // CHUNK
    dup_call, v2 = _build_dup(vocab)
    tt = table.T
    t4 = dup_call(tt, tt)
    xi = x.reshape(NW, n_blocks, CHUNK).astype(jnp.int32)
    par = (xi >= v2).astype(jnp.int32)
    out = _build_gather(n_blocks, t4.shape[0])(xi - par * v2, par, t4)
    return out.reshape(n_batch, seq_len, D_MODEL)
